# baseline (speedup 1.0000x reference)
import functools

import jax
import jax.numpy as jnp
from jax import lax
from jax.experimental import pallas as pl
from jax.experimental.pallas import tpu as pltpu
from jax.experimental.pallas import tpu_sc as plsc

B, S, H = 4, 4096, 768
N = B * S
LN_EPS = 1e-05

_NC, _NS = 2, 16
_NW = _NC * _NS
_P = 4
_NP = N // _P
_RPW = _NP // _NW
_CHUNK = 64
_NCHUNK = _RPW // _CHUNK


def _sc_gather_body(idx_hbm, table_hbm, out_hbm, idx_v,
                    rows0, rows1, g0, g1, w0, w1):
    wid = lax.axis_index("s") * _NC + lax.axis_index("c")
    base = wid * _RPW
    pltpu.sync_copy(idx_hbm.at[pl.ds(base, _RPW)], idx_v)
    bufs = ((rows0, g0, w0), (rows1, g1, w1))
    gd = [None, None]
    wd = [None, None]
    gd[0] = pltpu.async_copy(
        table_hbm.at[idx_v.at[pl.ds(0, _CHUNK)]], rows0, g0)
    for c in range(_NCHUNK):
        p = c & 1
        rows, _, ws = bufs[p]
        gd[p].wait()
        if c + 1 < _NCHUNK:
            q = (c + 1) & 1
            if wd[q] is not None:
                wd[q].wait()
            gd[q] = pltpu.async_copy(
                table_hbm.at[idx_v.at[pl.ds((c + 1) * _CHUNK, _CHUNK)]],
                bufs[q][0], bufs[q][1])
        wd[p] = pltpu.async_copy(
            rows, out_hbm.at[pl.ds(base + c * _CHUNK, _CHUNK)], ws)
    for d in wd:
        if d is not None:
            d.wait()


@functools.cache
def _sc_gather():
    return functools.partial(
        pl.kernel,
        mesh=plsc.VectorSubcoreMesh(core_axis_name="c", subcore_axis_name="s"),
        out_type=jax.ShapeDtypeStruct((_NP, H), jnp.float32),
        scratch_types=[
            pltpu.VMEM((_RPW,), jnp.int32),
            pltpu.VMEM((_CHUNK, H), jnp.float32),
            pltpu.VMEM((_CHUNK, H), jnp.float32),
            pltpu.SemaphoreType.DMA,
            pltpu.SemaphoreType.DMA,
            pltpu.SemaphoreType.DMA,
            pltpu.SemaphoreType.DMA,
        ],
    )(_sc_gather_body)


_BLK = 1024
_BPP = _NP // _BLK


def _ln_body(x_ref, pos_ref, tt_ref, ttab_ref, s_ref, b_ref, o_ref):
    h = x_ref[...] + pos_ref[...]
    t0 = ttab_ref[0:1, :]
    t1 = ttab_ref[1:2, :]
    h = h + t0 + tt_ref[...].astype(jnp.float32) * (t1 - t0)
    mean = jnp.mean(h, axis=-1, keepdims=True)
    c = h - mean
    var = jnp.mean(c * c, axis=-1, keepdims=True)
    o_ref[...] = c * lax.rsqrt(var + LN_EPS) * s_ref[...] + b_ref[...]


def _ln_body_buf(buf_ref, x_ref, pos_ref, tt_ref, ttab_ref, s_ref, b_ref,
                 o_ref):
    del buf_ref
    _ln_body(x_ref, pos_ref, tt_ref, ttab_ref, s_ref, b_ref, o_ref)


def _tc_ln_part(p, buf, x, pos_rows, tt, ttab, s, b):
    data_specs = [
        pl.BlockSpec((_BLK, H), lambda i: (p * _BPP + i, 0)),
        pl.BlockSpec((_BLK, H), lambda i: (i, 0)),
        pl.BlockSpec((_BLK, 1), lambda i: (p * _BPP + i, 0)),
        pl.BlockSpec((2, H), lambda i: (0, 0)),
        pl.BlockSpec((1, H), lambda i: (0, 0)),
        pl.BlockSpec((1, H), lambda i: (0, 0)),
    ]
    args = (x, pos_rows, tt, ttab, s, b)
    if buf is None:
        body, in_specs, aliases = _ln_body, data_specs, {}
    else:
        body = _ln_body_buf
        in_specs = [pl.BlockSpec(memory_space=pltpu.MemorySpace.HBM)]
        in_specs += data_specs
        args = (buf,) + args
        aliases = {0: 0}
    return pl.pallas_call(
        body,
        grid=(_BPP,),
        in_specs=in_specs,
        out_specs=pl.BlockSpec((_BLK, H), lambda i: (p * _BPP + i, 0)),
        out_shape=jax.ShapeDtypeStruct((N, H), jnp.float32),
        input_output_aliases=aliases,
        compiler_params=pltpu.CompilerParams(
            dimension_semantics=("arbitrary",),
        ),
    )(*args)


def kernel(inputs_embeds, token_type_ids, position_ids, attention_mask,
           pos_table, type_table, ln_scale, ln_bias):
    del attention_mask
    x = inputs_embeds.reshape(N, H)
    pid = position_ids.reshape(N).astype(jnp.int32)
    tt = token_type_ids.reshape(N, 1).astype(jnp.int32)
    s = ln_scale.reshape(1, H)
    b = ln_bias.reshape(1, H)

    pos_parts = [_sc_gather()(pid[p * _NP:(p + 1) * _NP], pos_table)
                 for p in range(_P)]
    buf = None
    for p in range(_P):
        buf = _tc_ln_part(p, buf, x, pos_parts[p], tt, type_table, s, b)
    return buf.reshape(B, S, H)


# speedup vs baseline: 1.0113x; 1.0113x over previous
import functools

import jax
import jax.numpy as jnp
from jax import lax
from jax.experimental import pallas as pl
from jax.experimental.pallas import tpu as pltpu
from jax.experimental.pallas import tpu_sc as plsc

B, S, H = 4, 4096, 768
N = B * S
LN_EPS = 1e-05

_NC, _NS = 2, 16
_NW = _NC * _NS
_P = 2
_NP = N // _P
_RPW = _NP // _NW
_CHUNK = 64
_NCHUNK = _RPW // _CHUNK


def _sc_gather_body(idx_hbm, table_hbm, out_hbm, idx_v,
                    rows0, rows1, g0, g1, w0, w1):
    wid = lax.axis_index("s") * _NC + lax.axis_index("c")
    base = wid * _RPW
    pltpu.sync_copy(idx_hbm.at[pl.ds(base, _RPW)], idx_v)
    bufs = ((rows0, g0, w0), (rows1, g1, w1))
    gd = [None, None]
    wd = [None, None]
    gd[0] = pltpu.async_copy(
        table_hbm.at[idx_v.at[pl.ds(0, _CHUNK)]], rows0, g0)
    for c in range(_NCHUNK):
        p = c & 1
        rows, _, ws = bufs[p]
        gd[p].wait()
        if c + 1 < _NCHUNK:
            q = (c + 1) & 1
            if wd[q] is not None:
                wd[q].wait()
            gd[q] = pltpu.async_copy(
                table_hbm.at[idx_v.at[pl.ds((c + 1) * _CHUNK, _CHUNK)]],
                bufs[q][0], bufs[q][1])
        wd[p] = pltpu.async_copy(
            rows, out_hbm.at[pl.ds(base + c * _CHUNK, _CHUNK)], ws)
    for d in wd:
        if d is not None:
            d.wait()


@functools.cache
def _sc_gather():
    return functools.partial(
        pl.kernel,
        mesh=plsc.VectorSubcoreMesh(core_axis_name="c", subcore_axis_name="s"),
        out_type=jax.ShapeDtypeStruct((_NP, H), jnp.float32),
        scratch_types=[
            pltpu.VMEM((_RPW,), jnp.int32),
            pltpu.VMEM((_CHUNK, H), jnp.float32),
            pltpu.VMEM((_CHUNK, H), jnp.float32),
            pltpu.SemaphoreType.DMA,
            pltpu.SemaphoreType.DMA,
            pltpu.SemaphoreType.DMA,
            pltpu.SemaphoreType.DMA,
        ],
    )(_sc_gather_body)


_BLK = 1024
_BPP = _NP // _BLK


def _ln_body(x_ref, pos_ref, tt_ref, ttab_ref, s_ref, b_ref, o_ref):
    h = x_ref[...] + pos_ref[...]
    t0 = ttab_ref[0:1, :]
    t1 = ttab_ref[1:2, :]
    h = h + t0 + tt_ref[...].astype(jnp.float32) * (t1 - t0)
    mean = jnp.mean(h, axis=-1, keepdims=True)
    c = h - mean
    var = jnp.mean(c * c, axis=-1, keepdims=True)
    o_ref[...] = c * lax.rsqrt(var + LN_EPS) * s_ref[...] + b_ref[...]


def _ln_body_buf(buf_ref, x_ref, pos_ref, tt_ref, ttab_ref, s_ref, b_ref,
                 o_ref):
    del buf_ref
    _ln_body(x_ref, pos_ref, tt_ref, ttab_ref, s_ref, b_ref, o_ref)


def _tc_ln_part(p, buf, x, pos_rows, tt, ttab, s, b):
    data_specs = [
        pl.BlockSpec((_BLK, H), lambda i: (p * _BPP + i, 0)),
        pl.BlockSpec((_BLK, H), lambda i: (i, 0)),
        pl.BlockSpec((_BLK, 1), lambda i: (p * _BPP + i, 0)),
        pl.BlockSpec((2, H), lambda i: (0, 0)),
        pl.BlockSpec((1, H), lambda i: (0, 0)),
        pl.BlockSpec((1, H), lambda i: (0, 0)),
    ]
    args = (x, pos_rows, tt, ttab, s, b)
    if buf is None:
        body, in_specs, aliases = _ln_body, data_specs, {}
    else:
        body = _ln_body_buf
        in_specs = [pl.BlockSpec(memory_space=pltpu.MemorySpace.HBM)]
        in_specs += data_specs
        args = (buf,) + args
        aliases = {0: 0}
    return pl.pallas_call(
        body,
        grid=(_BPP,),
        in_specs=in_specs,
        out_specs=pl.BlockSpec((_BLK, H), lambda i: (p * _BPP + i, 0)),
        out_shape=jax.ShapeDtypeStruct((N, H), jnp.float32),
        input_output_aliases=aliases,
        compiler_params=pltpu.CompilerParams(
            dimension_semantics=("arbitrary",),
        ),
    )(*args)


def kernel(inputs_embeds, token_type_ids, position_ids, attention_mask,
           pos_table, type_table, ln_scale, ln_bias):
    del attention_mask
    x = inputs_embeds.reshape(N, H)
    pid = position_ids.reshape(N).astype(jnp.int32)
    tt = token_type_ids.reshape(N, 1).astype(jnp.int32)
    s = ln_scale.reshape(1, H)
    b = ln_bias.reshape(1, H)

    pos_parts = [_sc_gather()(pid[p * _NP:(p + 1) * _NP], pos_table)
                 for p in range(_P)]
    buf = None
    for p in range(_P):
        buf = _tc_ln_part(p, buf, x, pos_parts[p], tt, type_table, s, b)
    return buf.reshape(B, S, H)


# speedup vs baseline: 1.0330x; 1.0216x over previous
import functools

import jax
import jax.numpy as jnp
from jax import lax
from jax.experimental import pallas as pl
from jax.experimental.pallas import tpu as pltpu
from jax.experimental.pallas import tpu_sc as plsc

B, S, H = 4, 4096, 768
N = B * S
LN_EPS = 1e-05

_NC, _NS = 2, 16
_NW = _NC * _NS
_P = 1
_NP = N // _P
_RPW = _NP // _NW
_CHUNK = 64
_NCHUNK = _RPW // _CHUNK


def _sc_gather_body(idx_hbm, table_hbm, out_hbm, idx_v,
                    rows0, rows1, g0, g1, w0, w1):
    wid = lax.axis_index("s") * _NC + lax.axis_index("c")
    base = wid * _RPW
    pltpu.sync_copy(idx_hbm.at[pl.ds(base, _RPW)], idx_v)
    bufs = ((rows0, g0, w0), (rows1, g1, w1))
    gd = [None, None]
    wd = [None, None]
    gd[0] = pltpu.async_copy(
        table_hbm.at[idx_v.at[pl.ds(0, _CHUNK)]], rows0, g0)
    for c in range(_NCHUNK):
        p = c & 1
        rows, _, ws = bufs[p]
        gd[p].wait()
        if c + 1 < _NCHUNK:
            q = (c + 1) & 1
            if wd[q] is not None:
                wd[q].wait()
            gd[q] = pltpu.async_copy(
                table_hbm.at[idx_v.at[pl.ds((c + 1) * _CHUNK, _CHUNK)]],
                bufs[q][0], bufs[q][1])
        wd[p] = pltpu.async_copy(
            rows, out_hbm.at[pl.ds(base + c * _CHUNK, _CHUNK)], ws)
    for d in wd:
        if d is not None:
            d.wait()


@functools.cache
def _sc_gather():
    return functools.partial(
        pl.kernel,
        mesh=plsc.VectorSubcoreMesh(core_axis_name="c", subcore_axis_name="s"),
        out_type=jax.ShapeDtypeStruct((_NP, H), jnp.float32),
        scratch_types=[
            pltpu.VMEM((_RPW,), jnp.int32),
            pltpu.VMEM((_CHUNK, H), jnp.float32),
            pltpu.VMEM((_CHUNK, H), jnp.float32),
            pltpu.SemaphoreType.DMA,
            pltpu.SemaphoreType.DMA,
            pltpu.SemaphoreType.DMA,
            pltpu.SemaphoreType.DMA,
        ],
    )(_sc_gather_body)


_BLK = 2048
_BPP = _NP // _BLK


def _ln_body(x_ref, pos_ref, tt_ref, ttab_ref, s_ref, b_ref, o_ref):
    h = x_ref[...] + pos_ref[...]
    t0 = ttab_ref[0:1, :]
    t1 = ttab_ref[1:2, :]
    h = h + t0 + tt_ref[...].astype(jnp.float32) * (t1 - t0)
    mean = jnp.mean(h, axis=-1, keepdims=True)
    c = h - mean
    var = jnp.mean(c * c, axis=-1, keepdims=True)
    o_ref[...] = c * lax.rsqrt(var + LN_EPS) * s_ref[...] + b_ref[...]


def _ln_body_buf(buf_ref, x_ref, pos_ref, tt_ref, ttab_ref, s_ref, b_ref,
                 o_ref):
    del buf_ref
    _ln_body(x_ref, pos_ref, tt_ref, ttab_ref, s_ref, b_ref, o_ref)


def _tc_ln_part(p, buf, x, pos_rows, tt, ttab, s, b):
    data_specs = [
        pl.BlockSpec((_BLK, H), lambda i: (p * _BPP + i, 0)),
        pl.BlockSpec((_BLK, H), lambda i: (i, 0)),
        pl.BlockSpec((_BLK, 1), lambda i: (p * _BPP + i, 0)),
        pl.BlockSpec((2, H), lambda i: (0, 0)),
        pl.BlockSpec((1, H), lambda i: (0, 0)),
        pl.BlockSpec((1, H), lambda i: (0, 0)),
    ]
    args = (x, pos_rows, tt, ttab, s, b)
    if buf is None:
        body, in_specs, aliases = _ln_body, data_specs, {}
    else:
        body = _ln_body_buf
        in_specs = [pl.BlockSpec(memory_space=pltpu.MemorySpace.HBM)]
        in_specs += data_specs
        args = (buf,) + args
        aliases = {0: 0}
    return pl.pallas_call(
        body,
        grid=(_BPP,),
        in_specs=in_specs,
        out_specs=pl.BlockSpec((_BLK, H), lambda i: (p * _BPP + i, 0)),
        out_shape=jax.ShapeDtypeStruct((N, H), jnp.float32),
        input_output_aliases=aliases,
        compiler_params=pltpu.CompilerParams(
            dimension_semantics=("arbitrary",),
        ),
    )(*args)


def kernel(inputs_embeds, token_type_ids, position_ids, attention_mask,
           pos_table, type_table, ln_scale, ln_bias):
    del attention_mask
    x = inputs_embeds.reshape(N, H)
    pid = position_ids.reshape(N).astype(jnp.int32)
    tt = token_type_ids.reshape(N, 1).astype(jnp.int32)
    s = ln_scale.reshape(1, H)
    b = ln_bias.reshape(1, H)

    pos_parts = [_sc_gather()(pid[p * _NP:(p + 1) * _NP], pos_table)
                 for p in range(_P)]
    buf = None
    for p in range(_P):
        buf = _tc_ln_part(p, buf, x, pos_parts[p], tt, type_table, s, b)
    return buf.reshape(B, S, H)
